# Initial kernel scaffold; baseline (speedup 1.0000x reference)
import functools

import jax
import jax.numpy as jnp
from jax import lax
from jax.experimental import pallas as pl
from jax.experimental.pallas import tpu as pltpu
from jax.experimental.pallas import tpu_sc as plsc

N = 10000
E = 320000
D = 128
ED = 16
H = 128
L = 3
OUT = 6
SPLIT = 3

NC = 2
NS = 16
NW = NC * NS
EPW = E // NW
CH = 80
NCHUNK = EPW // CH
ROWS_PER_SUB = N // NS

EB = 2000


def _edge_mlp_block(ea_ref, w_ref, b_ref, o_ref):
    a = ea_ref[...]
    e = jnp.dot(a, w_ref[...], preferred_element_type=jnp.float32)
    o_ref[...] = e + b_ref[...]


def _edge_mlp(edge_attr, We, be):
    return pl.pallas_call(
        _edge_mlp_block,
        grid=(E // EB,),
        in_specs=[
            pl.BlockSpec((EB, ED), lambda i: (i, 0)),
            pl.BlockSpec((ED, H), lambda i: (0, 0)),
            pl.BlockSpec((1, H), lambda i: (0, 0)),
        ],
        out_specs=pl.BlockSpec((EB, H), lambda i: (i, 0)),
        out_shape=jax.ShapeDtypeStruct((E, H), jnp.float32),
    )(edge_attr, We, be.reshape(1, H))


def _sc_aggregate(h, e, src, dst, zeros):
    mesh = plsc.VectorSubcoreMesh(core_axis_name="c", subcore_axis_name="s")

    @functools.partial(
        pl.kernel,
        out_type=jax.ShapeDtypeStruct((NC, N, H), jnp.float32),
        mesh=mesh,
        scratch_types=[
            pltpu.VMEM((CH,), jnp.int32),
            pltpu.VMEM((CH,), jnp.int32),
            pltpu.VMEM((CH, H), jnp.float32),
            pltpu.VMEM_SHARED((N, H), jnp.float32),
            pltpu.SemaphoreType.DMA,
        ],
    )
    def k(h_hbm, e_hbm, src_hbm, dst_hbm, z_hbm, out_hbm,
          sidx, didx, ebuf, aggr_sh, sem):
        c = lax.axis_index("c")
        s = lax.axis_index("s")
        w = c * NS + s
        pltpu.sync_copy(z_hbm.at[pl.ds(s * ROWS_PER_SUB, ROWS_PER_SUB)],
                        aggr_sh.at[pl.ds(s * ROWS_PER_SUB, ROWS_PER_SUB)])
        plsc.subcore_barrier()

        @pl.loop(0, NCHUNK)
        def _chunk(ci):
            base = w * EPW + ci * CH
            pltpu.sync_copy(src_hbm.at[pl.ds(base, CH)], sidx)
            pltpu.sync_copy(dst_hbm.at[pl.ds(base, CH)], didx)
            pltpu.sync_copy(e_hbm.at[pl.ds(base, CH)], ebuf)
            pltpu.async_copy(h_hbm.at[sidx], ebuf, sem, add=True).wait()

            @pl.loop(0, CH)
            def _relu(r):
                for j in range(8):
                    v = ebuf[r, pl.ds(j * 16, 16)]
                    ebuf[r, pl.ds(j * 16, 16)] = jnp.maximum(v, 0.0)

            pltpu.sync_copy(ebuf, aggr_sh.at[didx], add=True)

        plsc.subcore_barrier()
        pltpu.sync_copy(aggr_sh.at[pl.ds(s * ROWS_PER_SUB, ROWS_PER_SUB)],
                        out_hbm.at[c].at[pl.ds(s * ROWS_PER_SUB, ROWS_PER_SUB)])

    return k(h, e, src, dst, zeros)


def _node_block(h_ref, a0_ref, a1_ref, w1_ref, b1_ref, w2_ref, b2_ref,
                gnw_ref, gnb_ref, gnms_ref, o_ref):
    h = h_ref[...]
    z0 = h + a0_ref[...] + a1_ref[...]
    t = jnp.maximum(
        jnp.dot(z0, w1_ref[...], preferred_element_type=jnp.float32)
        + b1_ref[...], 0.0)
    t = jnp.dot(t, w2_ref[...], preferred_element_type=jnp.float32) + b2_ref[...]
    mean = jnp.mean(t, axis=0, keepdims=True)
    cen = t - gnms_ref[...] * mean
    var = jnp.mean(cen * cen, axis=0, keepdims=True)
    zn = gnw_ref[...] * cen * lax.rsqrt(var + 1e-5) + gnb_ref[...]
    o_ref[...] = jnp.maximum(zn, 0.0) + h


def _node_update(h, aggr2, lp):
    return pl.pallas_call(
        _node_block,
        out_shape=jax.ShapeDtypeStruct((N, H), jnp.float32),
    )(h, aggr2[0], aggr2[1],
      lp["W1"], lp["b1"].reshape(1, H), lp["W2"], lp["b2"].reshape(1, H),
      lp["gn_w"].reshape(1, H), lp["gn_b"].reshape(1, H),
      lp["gn_ms"].reshape(1, H))


def _head_block(z1_ref, z2_ref, z3_ref, wh1_ref, bh1_ref, wh2_ref, bh2_ref,
                o_ref):
    w = wh1_ref[...]
    t = (jnp.dot(z1_ref[...], w[0:H], preferred_element_type=jnp.float32)
         + jnp.dot(z2_ref[...], w[H:2 * H], preferred_element_type=jnp.float32)
         + jnp.dot(z3_ref[...], w[2 * H:3 * H],
                   preferred_element_type=jnp.float32))
    t = jnp.maximum(t + bh1_ref[...], 0.0)
    o_ref[...] = (jnp.dot(t, wh2_ref[...], preferred_element_type=jnp.float32)
                  + bh2_ref[...])


def _head(z1, z2, z3, Wh1, bh1, Wh2, bh2):
    wh2p = jnp.zeros((H, 8), jnp.float32).at[:, :OUT].set(Wh2)
    bh2p = jnp.zeros((1, 8), jnp.float32).at[0, :OUT].set(bh2)
    return pl.pallas_call(
        _head_block,
        out_shape=jax.ShapeDtypeStruct((N, 8), jnp.float32),
    )(z1, z2, z3, Wh1, bh1.reshape(1, H), wh2p, bh2p)


def kernel(x, edge_index, edge_attr, params):
    src = edge_index[0]
    dst = edge_index[1]
    zeros = jnp.zeros((N, H), jnp.float32)

    es = [_edge_mlp(edge_attr, lp["We"], lp["be"]) for lp in params["layers"]]

    h = x
    outs = []
    for l, lp in enumerate(params["layers"]):
        aggr2 = _sc_aggregate(h, es[l], src, dst, zeros)
        h = _node_update(h, aggr2, lp)
        outs.append(h)

    head = _head(outs[0], outs[1], outs[2],
                 params["Wh1"], params["bh1"], params["Wh2"], params["bh2"])
    return head[:, :SPLIT], head[:, SPLIT:OUT]


# speedup vs baseline: 2.5968x; 2.5968x over previous
import functools

import jax
import jax.numpy as jnp
from jax import lax
from jax.experimental import pallas as pl
from jax.experimental.pallas import tpu as pltpu
from jax.experimental.pallas import tpu_sc as plsc

N = 10000
E = 320000
D = 128
ED = 16
H = 128
L = 3
OUT = 6
SPLIT = 3

NC = 2
NS = 16
NW = NC * NS
EPW = E // NW
CH = 80
NCHUNK = EPW // CH
RSUB = 624
RTAIL = N - NS * RSUB

EB = 2000


def _edge_mlp_block(ea_ref, w_ref, b_ref, o_ref):
    a = ea_ref[...]
    e = jnp.dot(a, w_ref[...], preferred_element_type=jnp.float32)
    o_ref[...] = e + b_ref[...]


def _edge_mlp(edge_attr, We, be):
    return pl.pallas_call(
        _edge_mlp_block,
        grid=(E // EB,),
        in_specs=[
            pl.BlockSpec((EB, ED), lambda i: (i, 0)),
            pl.BlockSpec((ED, H), lambda i: (0, 0)),
            pl.BlockSpec((1, H), lambda i: (0, 0)),
        ],
        out_specs=pl.BlockSpec((EB, H), lambda i: (i, 0)),
        out_shape=jax.ShapeDtypeStruct((E, H), jnp.float32),
    )(edge_attr, We, be.reshape(1, H))


def _sc_aggregate(h, e, src, dst, zeros):
    mesh = plsc.VectorSubcoreMesh(core_axis_name="c", subcore_axis_name="s")

    @functools.partial(
        pl.kernel,
        out_type=jax.ShapeDtypeStruct((NC, N, H), jnp.float32),
        mesh=mesh,
        scratch_types=[
            pltpu.VMEM((CH,), jnp.int32),
            pltpu.VMEM((CH,), jnp.int32),
            pltpu.VMEM((CH, H), jnp.float32),
            pltpu.VMEM_SHARED((N, H), jnp.float32),
            pltpu.SemaphoreType.DMA,
        ],
    )
    def k(h_hbm, e_hbm, src_hbm, dst_hbm, z_hbm, out_hbm,
          sidx, didx, ebuf, aggr_sh, sem):
        c = lax.axis_index("c")
        s = lax.axis_index("s")
        w = c * NS + s
        pltpu.sync_copy(z_hbm.at[pl.ds(s * RSUB, RSUB)],
                        aggr_sh.at[pl.ds(s * RSUB, RSUB)])

        @pl.when(s == 0)
        def _():
            pltpu.sync_copy(z_hbm.at[pl.ds(NS * RSUB, RTAIL)],
                            aggr_sh.at[pl.ds(NS * RSUB, RTAIL)])

        plsc.subcore_barrier()

        @pl.loop(0, NCHUNK)
        def _chunk(ci):
            base = w * EPW + ci * CH
            pltpu.sync_copy(src_hbm.at[pl.ds(base, CH)], sidx)
            pltpu.sync_copy(dst_hbm.at[pl.ds(base, CH)], didx)
            pltpu.sync_copy(e_hbm.at[pl.ds(base, CH)], ebuf)
            pltpu.async_copy(h_hbm.at[sidx], ebuf, sem, add=True).wait()

            @pl.loop(0, CH)
            def _relu(r):
                for j in range(8):
                    v = ebuf[r, pl.ds(j * 16, 16)]
                    ebuf[r, pl.ds(j * 16, 16)] = jnp.maximum(v, 0.0)

            pltpu.sync_copy(ebuf, aggr_sh.at[didx], add=True)

        plsc.subcore_barrier()
        pltpu.sync_copy(aggr_sh.at[pl.ds(s * RSUB, RSUB)],
                        out_hbm.at[c].at[pl.ds(s * RSUB, RSUB)])

        @pl.when(s == 0)
        def _():
            pltpu.sync_copy(aggr_sh.at[pl.ds(NS * RSUB, RTAIL)],
                            out_hbm.at[c].at[pl.ds(NS * RSUB, RTAIL)])

    return k(h, e, src, dst, zeros)


def _node_block(h_ref, a0_ref, a1_ref, w1_ref, b1_ref, w2_ref, b2_ref,
                gnw_ref, gnb_ref, gnms_ref, o_ref):
    h = h_ref[...]
    z0 = h + a0_ref[...] + a1_ref[...]
    t = jnp.maximum(
        jnp.dot(z0, w1_ref[...], preferred_element_type=jnp.float32)
        + b1_ref[...], 0.0)
    t = jnp.dot(t, w2_ref[...], preferred_element_type=jnp.float32) + b2_ref[...]
    mean = jnp.mean(t, axis=0, keepdims=True)
    cen = t - gnms_ref[...] * mean
    var = jnp.mean(cen * cen, axis=0, keepdims=True)
    zn = gnw_ref[...] * cen * lax.rsqrt(var + 1e-5) + gnb_ref[...]
    o_ref[...] = jnp.maximum(zn, 0.0) + h


def _node_update(h, aggr2, lp):
    return pl.pallas_call(
        _node_block,
        out_shape=jax.ShapeDtypeStruct((N, H), jnp.float32),
    )(h, aggr2[0], aggr2[1],
      lp["W1"], lp["b1"].reshape(1, H), lp["W2"], lp["b2"].reshape(1, H),
      lp["gn_w"].reshape(1, H), lp["gn_b"].reshape(1, H),
      lp["gn_ms"].reshape(1, H))


def _head_block(z1_ref, z2_ref, z3_ref, wh1_ref, bh1_ref, wh2_ref, bh2_ref,
                o_ref):
    w = wh1_ref[...]
    t = (jnp.dot(z1_ref[...], w[0:H], preferred_element_type=jnp.float32)
         + jnp.dot(z2_ref[...], w[H:2 * H], preferred_element_type=jnp.float32)
         + jnp.dot(z3_ref[...], w[2 * H:3 * H],
                   preferred_element_type=jnp.float32))
    t = jnp.maximum(t + bh1_ref[...], 0.0)
    o_ref[...] = (jnp.dot(t, wh2_ref[...], preferred_element_type=jnp.float32)
                  + bh2_ref[...])


def _head(z1, z2, z3, Wh1, bh1, Wh2, bh2):
    wh2p = jnp.zeros((H, 8), jnp.float32).at[:, :OUT].set(Wh2)
    bh2p = jnp.zeros((1, 8), jnp.float32).at[0, :OUT].set(bh2)
    return pl.pallas_call(
        _head_block,
        out_shape=jax.ShapeDtypeStruct((N, 8), jnp.float32),
    )(z1, z2, z3, Wh1, bh1.reshape(1, H), wh2p, bh2p)


def kernel(x, edge_index, edge_attr, params):
    src = edge_index[0]
    dst = edge_index[1]
    zeros = jnp.zeros((N, H), jnp.float32)

    es = [_edge_mlp(edge_attr, lp["We"], lp["be"]) for lp in params["layers"]]

    h = x
    outs = []
    for l, lp in enumerate(params["layers"]):
        aggr2 = _sc_aggregate(h, es[l], src, dst, zeros)
        h = _node_update(h, aggr2, lp)
        outs.append(h)

    head = _head(outs[0], outs[1], outs[2],
                 params["Wh1"], params["bh1"], params["Wh2"], params["bh2"])
    return head[:, :SPLIT], head[:, SPLIT:OUT]


# speedup vs baseline: 4.0881x; 1.5743x over previous
import functools

import jax
import jax.numpy as jnp
from jax import lax
from jax.experimental import pallas as pl
from jax.experimental.pallas import tpu as pltpu
from jax.experimental.pallas import tpu_sc as plsc

N = 10000
E = 320000
D = 128
ED = 16
H = 128
L = 3
OUT = 6
SPLIT = 3

NC = 2
NS = 16
NW = NC * NS
EPW = E // NW
CH = 80
G = 2
SCH = G * CH
NITER = EPW // SCH
TAIL = EPW - NITER * SCH
RSUB = 624
RTAIL = N - NS * RSUB

EB = 2000


def _edge_mlp_block(ea_ref, w_ref, b_ref, o_ref):
    a = ea_ref[...]
    e = jnp.dot(a, w_ref[...], preferred_element_type=jnp.float32)
    o_ref[...] = e + b_ref[...]


def _edge_mlp(edge_attr, We, be):
    return pl.pallas_call(
        _edge_mlp_block,
        grid=(E // EB,),
        in_specs=[
            pl.BlockSpec((EB, ED), lambda i: (i, 0)),
            pl.BlockSpec((ED, H), lambda i: (0, 0)),
            pl.BlockSpec((1, H), lambda i: (0, 0)),
        ],
        out_specs=pl.BlockSpec((EB, H), lambda i: (i, 0)),
        out_shape=jax.ShapeDtypeStruct((E, H), jnp.float32),
    )(edge_attr, We, be.reshape(1, H))


def _sc_aggregate(h, e, src, dst, zeros):
    mesh = plsc.VectorSubcoreMesh(core_axis_name="c", subcore_axis_name="s")

    @functools.partial(
        pl.kernel,
        out_type=jax.ShapeDtypeStruct((NC, N, H), jnp.float32),
        mesh=mesh,
        scratch_types=[
            pltpu.VMEM((SCH,), jnp.int32),
            pltpu.VMEM((SCH,), jnp.int32),
            pltpu.VMEM((G, CH), jnp.int32),
            pltpu.VMEM((G, CH), jnp.int32),
            pltpu.VMEM((SCH, H), jnp.float32),
            pltpu.VMEM((SCH, H), jnp.float32),
            pltpu.VMEM_SHARED((N, H), jnp.float32),
            pltpu.SemaphoreType.DMA,
            pltpu.SemaphoreType.DMA,
            pltpu.SemaphoreType.DMA,
            pltpu.SemaphoreType.DMA,
        ],
    )
    def k(h_hbm, e_hbm, src_hbm, dst_hbm, z_hbm, out_hbm,
          sidx0, sidx1, didx0, didx1, ebuf0, ebuf1, aggr_sh,
          sem_l0, sem_l1, sem_g, sem_s):
        c = lax.axis_index("c")
        s = lax.axis_index("s")
        w = c * NS + s
        sidxs = (sidx0, sidx1)
        didxs = (didx0, didx1)
        ebufs = (ebuf0, ebuf1)
        sems = (sem_l0, sem_l1)
        pltpu.sync_copy(z_hbm.at[pl.ds(s * RSUB, RSUB)],
                        aggr_sh.at[pl.ds(s * RSUB, RSUB)])

        @pl.when(s == 0)
        def _():
            pltpu.sync_copy(z_hbm.at[pl.ds(NS * RSUB, RTAIL)],
                            aggr_sh.at[pl.ds(NS * RSUB, RTAIL)])

        plsc.subcore_barrier()

        def issue_linear(it, b):
            base = w * EPW + it * SCH
            pltpu.async_copy(e_hbm.at[pl.ds(base, SCH)], ebufs[b], sems[b])
            pltpu.async_copy(src_hbm.at[pl.ds(base, SCH)], sidxs[b], sems[b])
            for g in range(G):
                pltpu.async_copy(dst_hbm.at[pl.ds(base + g * CH, CH)],
                                 didxs[b].at[g], sems[b])

        def drain_linear(it, b):
            base = w * EPW + it * SCH
            pltpu.make_async_copy(e_hbm.at[pl.ds(base, SCH)],
                                  ebufs[b], sems[b]).wait()
            pltpu.make_async_copy(src_hbm.at[pl.ds(base, SCH)],
                                  sidxs[b], sems[b]).wait()
            for g in range(G):
                pltpu.make_async_copy(dst_hbm.at[pl.ds(base + g * CH, CH)],
                                      didxs[b].at[g], sems[b]).wait()

        def consume(it, b):
            drain_linear(it, b)
            gds = [pltpu.async_copy(
                h_hbm.at[sidxs[b].at[pl.ds(g * CH, CH)]],
                ebufs[b].at[pl.ds(g * CH, CH)], sem_g, add=True)
                for g in range(G)]
            for gd in gds:
                gd.wait()

            @pl.loop(0, SCH)
            def _relu(r):
                for j in range(8):
                    v = ebufs[b][r, pl.ds(j * 16, 16)]
                    ebufs[b][r, pl.ds(j * 16, 16)] = jnp.maximum(v, 0.0)

            sds = [pltpu.async_copy(
                ebufs[b].at[pl.ds(g * CH, CH)],
                aggr_sh.at[didxs[b].at[g]], sem_s, add=True)
                for g in range(G)]
            for sd in sds:
                sd.wait()

        issue_linear(0, 0)

        @pl.loop(0, NITER, step=2)
        def _outer(it):
            issue_linear(it + 1, 1)
            consume(it, 0)

            @pl.when(it + 2 < NITER)
            def _():
                issue_linear(it + 2, 0)

            consume(it + 1, 1)

        tbase = w * EPW + NITER * SCH
        pltpu.sync_copy(src_hbm.at[pl.ds(tbase, TAIL)],
                        sidx0.at[pl.ds(0, TAIL)])
        pltpu.sync_copy(dst_hbm.at[pl.ds(tbase, TAIL)], didx0.at[0])
        pltpu.sync_copy(e_hbm.at[pl.ds(tbase, TAIL)],
                        ebuf0.at[pl.ds(0, TAIL)])
        pltpu.async_copy(h_hbm.at[sidx0.at[pl.ds(0, TAIL)]],
                         ebuf0.at[pl.ds(0, TAIL)], sem_g, add=True).wait()

        @pl.loop(0, TAIL)
        def _relu_tail(r):
            for j in range(8):
                v = ebuf0[r, pl.ds(j * 16, 16)]
                ebuf0[r, pl.ds(j * 16, 16)] = jnp.maximum(v, 0.0)

        pltpu.sync_copy(ebuf0.at[pl.ds(0, TAIL)],
                        aggr_sh.at[didx0.at[0]], add=True)

        plsc.subcore_barrier()
        pltpu.sync_copy(aggr_sh.at[pl.ds(s * RSUB, RSUB)],
                        out_hbm.at[c].at[pl.ds(s * RSUB, RSUB)])

        @pl.when(s == 0)
        def _():
            pltpu.sync_copy(aggr_sh.at[pl.ds(NS * RSUB, RTAIL)],
                            out_hbm.at[c].at[pl.ds(NS * RSUB, RTAIL)])

    return k(h, e, src, dst, zeros)


def _node_block(h_ref, a0_ref, a1_ref, w1_ref, b1_ref, w2_ref, b2_ref,
                gnw_ref, gnb_ref, gnms_ref, o_ref):
    h = h_ref[...]
    z0 = h + a0_ref[...] + a1_ref[...]
    t = jnp.maximum(
        jnp.dot(z0, w1_ref[...], preferred_element_type=jnp.float32)
        + b1_ref[...], 0.0)
    t = jnp.dot(t, w2_ref[...], preferred_element_type=jnp.float32) + b2_ref[...]
    mean = jnp.mean(t, axis=0, keepdims=True)
    cen = t - gnms_ref[...] * mean
    var = jnp.mean(cen * cen, axis=0, keepdims=True)
    zn = gnw_ref[...] * cen * lax.rsqrt(var + 1e-5) + gnb_ref[...]
    o_ref[...] = jnp.maximum(zn, 0.0) + h


def _node_update(h, aggr2, lp):
    return pl.pallas_call(
        _node_block,
        out_shape=jax.ShapeDtypeStruct((N, H), jnp.float32),
    )(h, aggr2[0], aggr2[1],
      lp["W1"], lp["b1"].reshape(1, H), lp["W2"], lp["b2"].reshape(1, H),
      lp["gn_w"].reshape(1, H), lp["gn_b"].reshape(1, H),
      lp["gn_ms"].reshape(1, H))


def _head_block(z1_ref, z2_ref, z3_ref, wh1_ref, bh1_ref, wh2_ref, bh2_ref,
                o_ref):
    w = wh1_ref[...]
    t = (jnp.dot(z1_ref[...], w[0:H], preferred_element_type=jnp.float32)
         + jnp.dot(z2_ref[...], w[H:2 * H], preferred_element_type=jnp.float32)
         + jnp.dot(z3_ref[...], w[2 * H:3 * H],
                   preferred_element_type=jnp.float32))
    t = jnp.maximum(t + bh1_ref[...], 0.0)
    o_ref[...] = (jnp.dot(t, wh2_ref[...], preferred_element_type=jnp.float32)
                  + bh2_ref[...])


def _head(z1, z2, z3, Wh1, bh1, Wh2, bh2):
    wh2p = jnp.zeros((H, 8), jnp.float32).at[:, :OUT].set(Wh2)
    bh2p = jnp.zeros((1, 8), jnp.float32).at[0, :OUT].set(bh2)
    return pl.pallas_call(
        _head_block,
        out_shape=jax.ShapeDtypeStruct((N, 8), jnp.float32),
    )(z1, z2, z3, Wh1, bh1.reshape(1, H), wh2p, bh2p)


def kernel(x, edge_index, edge_attr, params):
    src = edge_index[0]
    dst = edge_index[1]
    zeros = jnp.zeros((N, H), jnp.float32)

    es = [_edge_mlp(edge_attr, lp["We"], lp["be"]) for lp in params["layers"]]

    h = x
    outs = []
    for l, lp in enumerate(params["layers"]):
        aggr2 = _sc_aggregate(h, es[l], src, dst, zeros)
        h = _node_update(h, aggr2, lp)
        outs.append(h)

    head = _head(outs[0], outs[1], outs[2],
                 params["Wh1"], params["bh1"], params["Wh2"], params["bh2"])
    return head[:, :SPLIT], head[:, SPLIT:OUT]


# speedup vs baseline: 4.8975x; 1.1980x over previous
import functools

import jax
import jax.numpy as jnp
from jax import lax
from jax.experimental import pallas as pl
from jax.experimental.pallas import tpu as pltpu
from jax.experimental.pallas import tpu_sc as plsc

N = 10000
E = 320000
D = 128
ED = 16
H = 128
L = 3
OUT = 6
SPLIT = 3

NC = 2
NS = 16
NW = NC * NS
EPW = E // NW
CH = 80
NB = 4
NSTEP = EPW // CH
RSUB = 624
RTAIL = N - NS * RSUB

EB = 2000


def _edge_mlp_block(ea_ref, w_ref, b_ref, o_ref):
    a = ea_ref[...]
    e = jnp.dot(a, w_ref[...], preferred_element_type=jnp.float32)
    o_ref[...] = e + b_ref[...]


def _edge_mlp(edge_attr, We, be):
    return pl.pallas_call(
        _edge_mlp_block,
        grid=(E // EB,),
        in_specs=[
            pl.BlockSpec((EB, ED), lambda i: (i, 0)),
            pl.BlockSpec((ED, H), lambda i: (0, 0)),
            pl.BlockSpec((1, H), lambda i: (0, 0)),
        ],
        out_specs=pl.BlockSpec((EB, H), lambda i: (i, 0)),
        out_shape=jax.ShapeDtypeStruct((E, H), jnp.float32),
    )(edge_attr, We, be.reshape(1, H))


def _sc_aggregate(h, e, src, dst, zeros):
    mesh = plsc.VectorSubcoreMesh(core_axis_name="c", subcore_axis_name="s")

    @functools.partial(
        pl.kernel,
        out_type=jax.ShapeDtypeStruct((NC, N, H), jnp.float32),
        mesh=mesh,
        scratch_types=(
            [pltpu.VMEM((CH,), jnp.int32) for _ in range(NB)]
            + [pltpu.VMEM((CH,), jnp.int32) for _ in range(NB)]
            + [pltpu.VMEM((CH, H), jnp.float32) for _ in range(NB)]
            + [pltpu.VMEM_SHARED((N, H), jnp.float32)]
            + [pltpu.SemaphoreType.DMA for _ in range(NB)]
        ),
    )
    def k(h_hbm, e_hbm, src_hbm, dst_hbm, z_hbm, out_hbm,
          s0, s1, s2, s3, d0, d1, d2, d3, e0, e1, e2, e3, aggr_sh,
          m0, m1, m2, m3):
        c = lax.axis_index("c")
        s = lax.axis_index("s")
        w = c * NS + s
        sidxs = (s0, s1, s2, s3)
        didxs = (d0, d1, d2, d3)
        ebufs = (e0, e1, e2, e3)
        sems = (m0, m1, m2, m3)
        pltpu.sync_copy(z_hbm.at[pl.ds(s * RSUB, RSUB)],
                        aggr_sh.at[pl.ds(s * RSUB, RSUB)])

        @pl.when(s == 0)
        def _():
            pltpu.sync_copy(z_hbm.at[pl.ds(NS * RSUB, RTAIL)],
                            aggr_sh.at[pl.ds(NS * RSUB, RTAIL)])

        plsc.subcore_barrier()

        def L(i, b, guard_scatter):
            base = w * EPW + i * CH

            def drain():
                pltpu.make_async_copy(ebufs[b], aggr_sh.at[didxs[b]],
                                      sems[b]).wait()

            if guard_scatter:
                pl.when(i >= NB)(drain)
            else:
                drain()
            pltpu.async_copy(e_hbm.at[pl.ds(base, CH)], ebufs[b], sems[b])
            pltpu.async_copy(src_hbm.at[pl.ds(base, CH)], sidxs[b], sems[b])
            pltpu.async_copy(dst_hbm.at[pl.ds(base, CH)], didxs[b], sems[b])

        def Lfirst(i, b):
            base = w * EPW + i * CH
            pltpu.async_copy(e_hbm.at[pl.ds(base, CH)], ebufs[b], sems[b])
            pltpu.async_copy(src_hbm.at[pl.ds(base, CH)], sidxs[b], sems[b])
            pltpu.async_copy(dst_hbm.at[pl.ds(base, CH)], didxs[b], sems[b])

        def G(i, b):
            base = w * EPW + i * CH
            pltpu.make_async_copy(e_hbm.at[pl.ds(base, CH)],
                                  ebufs[b], sems[b]).wait()
            pltpu.make_async_copy(src_hbm.at[pl.ds(base, CH)],
                                  sidxs[b], sems[b]).wait()
            pltpu.make_async_copy(dst_hbm.at[pl.ds(base, CH)],
                                  didxs[b], sems[b]).wait()
            pltpu.async_copy(h_hbm.at[sidxs[b]], ebufs[b], sems[b], add=True)

        def C(b):
            pltpu.make_async_copy(h_hbm.at[sidxs[b]], ebufs[b],
                                  sems[b]).wait()

            @pl.loop(0, CH)
            def _relu(r):
                for j in range(8):
                    v = ebufs[b][r, pl.ds(j * 16, 16)]
                    ebufs[b][r, pl.ds(j * 16, 16)] = jnp.maximum(v, 0.0)

            pltpu.async_copy(ebufs[b], aggr_sh.at[didxs[b]], sems[b],
                             add=True)

        Lfirst(0, 0)
        Lfirst(1, 1)
        G(0, 0)

        @pl.loop(0, NSTEP - 1, step=NB)
        def _outer(it):
            for o in range(NB):
                i = it + o

                @pl.when(i + 2 < NSTEP)
                def _(i=i, o=o):
                    L(i + 2, (o + 2) % NB, guard_scatter=True)

                G(i + 1, (o + 1) % NB)
                C(o)

        C((NSTEP - 1) % NB)
        for b in range(NB):
            pltpu.make_async_copy(ebufs[b], aggr_sh.at[didxs[b]],
                                  sems[b]).wait()

        plsc.subcore_barrier()
        pltpu.sync_copy(aggr_sh.at[pl.ds(s * RSUB, RSUB)],
                        out_hbm.at[c].at[pl.ds(s * RSUB, RSUB)])

        @pl.when(s == 0)
        def _():
            pltpu.sync_copy(aggr_sh.at[pl.ds(NS * RSUB, RTAIL)],
                            out_hbm.at[c].at[pl.ds(NS * RSUB, RTAIL)])

    return k(h, e, src, dst, zeros)


def _node_block(h_ref, a0_ref, a1_ref, w1_ref, b1_ref, w2_ref, b2_ref,
                gnw_ref, gnb_ref, gnms_ref, o_ref):
    h = h_ref[...]
    z0 = h + a0_ref[...] + a1_ref[...]
    t = jnp.maximum(
        jnp.dot(z0, w1_ref[...], preferred_element_type=jnp.float32)
        + b1_ref[...], 0.0)
    t = jnp.dot(t, w2_ref[...], preferred_element_type=jnp.float32) + b2_ref[...]
    mean = jnp.mean(t, axis=0, keepdims=True)
    cen = t - gnms_ref[...] * mean
    var = jnp.mean(cen * cen, axis=0, keepdims=True)
    zn = gnw_ref[...] * cen * lax.rsqrt(var + 1e-5) + gnb_ref[...]
    o_ref[...] = jnp.maximum(zn, 0.0) + h


def _node_update(h, aggr2, lp):
    return pl.pallas_call(
        _node_block,
        out_shape=jax.ShapeDtypeStruct((N, H), jnp.float32),
    )(h, aggr2[0], aggr2[1],
      lp["W1"], lp["b1"].reshape(1, H), lp["W2"], lp["b2"].reshape(1, H),
      lp["gn_w"].reshape(1, H), lp["gn_b"].reshape(1, H),
      lp["gn_ms"].reshape(1, H))


def _head_block(z1_ref, z2_ref, z3_ref, wh1_ref, bh1_ref, wh2_ref, bh2_ref,
                o_ref):
    w = wh1_ref[...]
    t = (jnp.dot(z1_ref[...], w[0:H], preferred_element_type=jnp.float32)
         + jnp.dot(z2_ref[...], w[H:2 * H], preferred_element_type=jnp.float32)
         + jnp.dot(z3_ref[...], w[2 * H:3 * H],
                   preferred_element_type=jnp.float32))
    t = jnp.maximum(t + bh1_ref[...], 0.0)
    o_ref[...] = (jnp.dot(t, wh2_ref[...], preferred_element_type=jnp.float32)
                  + bh2_ref[...])


def _head(z1, z2, z3, Wh1, bh1, Wh2, bh2):
    wh2p = jnp.zeros((H, 8), jnp.float32).at[:, :OUT].set(Wh2)
    bh2p = jnp.zeros((1, 8), jnp.float32).at[0, :OUT].set(bh2)
    return pl.pallas_call(
        _head_block,
        out_shape=jax.ShapeDtypeStruct((N, 8), jnp.float32),
    )(z1, z2, z3, Wh1, bh1.reshape(1, H), wh2p, bh2p)


def kernel(x, edge_index, edge_attr, params):
    src = edge_index[0]
    dst = edge_index[1]
    zeros = jnp.zeros((N, H), jnp.float32)

    es = [_edge_mlp(edge_attr, lp["We"], lp["be"]) for lp in params["layers"]]

    h = x
    outs = []
    for l, lp in enumerate(params["layers"]):
        aggr2 = _sc_aggregate(h, es[l], src, dst, zeros)
        h = _node_update(h, aggr2, lp)
        outs.append(h)

    head = _head(outs[0], outs[1], outs[2],
                 params["Wh1"], params["bh1"], params["Wh2"], params["bh2"])
    return head[:, :SPLIT], head[:, SPLIT:OUT]


# speedup vs baseline: 5.0621x; 1.0336x over previous
import functools

import jax
import jax.numpy as jnp
from jax import lax
from jax.experimental import pallas as pl
from jax.experimental.pallas import tpu as pltpu
from jax.experimental.pallas import tpu_sc as plsc

N = 10000
E = 320000
D = 128
ED = 16
H = 128
L = 3
OUT = 6
SPLIT = 3

NC = 2
NS = 16
NW = NC * NS
EPW = E // NW
CH = 80
NB = 4
NSTEP = EPW // CH
RSUB = 624
RTAIL = N - NS * RSUB

EB = 2000


def _edge_mlp_block(ea_ref, w_ref, b_ref, o_ref):
    a = ea_ref[...]
    e = jnp.dot(a, w_ref[...], preferred_element_type=jnp.float32)
    o_ref[...] = e + b_ref[...]


def _edge_mlp(edge_attr, We, be):
    return pl.pallas_call(
        _edge_mlp_block,
        grid=(E // EB,),
        in_specs=[
            pl.BlockSpec((EB, ED), lambda i: (i, 0)),
            pl.BlockSpec((ED, H), lambda i: (0, 0)),
            pl.BlockSpec((1, H), lambda i: (0, 0)),
        ],
        out_specs=pl.BlockSpec((EB, H), lambda i: (i, 0)),
        out_shape=jax.ShapeDtypeStruct((E, H), jnp.float32),
    )(edge_attr, We, be.reshape(1, H))


def _sc_aggregate(h, e, edge_index, zeros):
    mesh = plsc.VectorSubcoreMesh(core_axis_name="c", subcore_axis_name="s")

    @functools.partial(
        pl.kernel,
        out_type=jax.ShapeDtypeStruct((NC, N, H), jnp.float32),
        mesh=mesh,
        scratch_types=(
            [pltpu.VMEM((CH,), jnp.int32) for _ in range(NB)]
            + [pltpu.VMEM((CH,), jnp.int32) for _ in range(NB)]
            + [pltpu.VMEM((CH, H), jnp.float32) for _ in range(NB)]
            + [pltpu.VMEM_SHARED((N, H), jnp.float32)]
            + [pltpu.SemaphoreType.DMA for _ in range(NB)]
        ),
    )
    def k(h_hbm, e_hbm, ei_hbm, z_hbm, out_hbm,
          s0, s1, s2, s3, d0, d1, d2, d3, e0, e1, e2, e3, aggr_sh,
          m0, m1, m2, m3):
        src_hbm = ei_hbm.at[pl.ds(0, E)]
        dst_hbm = ei_hbm.at[pl.ds(E, E)]
        c = lax.axis_index("c")
        s = lax.axis_index("s")
        w = c * NS + s
        sidxs = (s0, s1, s2, s3)
        didxs = (d0, d1, d2, d3)
        ebufs = (e0, e1, e2, e3)
        sems = (m0, m1, m2, m3)
        pltpu.sync_copy(z_hbm.at[pl.ds(s * RSUB, RSUB)],
                        aggr_sh.at[pl.ds(s * RSUB, RSUB)])

        @pl.when(s == 0)
        def _():
            pltpu.sync_copy(z_hbm.at[pl.ds(NS * RSUB, RTAIL)],
                            aggr_sh.at[pl.ds(NS * RSUB, RTAIL)])

        plsc.subcore_barrier()

        def L(i, b, guard_scatter):
            base = w * EPW + i * CH

            def drain():
                pltpu.make_async_copy(ebufs[b], aggr_sh.at[didxs[b]],
                                      sems[b]).wait()

            if guard_scatter:
                pl.when(i >= NB)(drain)
            else:
                drain()
            pltpu.async_copy(e_hbm.at[pl.ds(base, CH)], ebufs[b], sems[b])
            pltpu.async_copy(src_hbm.at[pl.ds(base, CH)], sidxs[b], sems[b])
            pltpu.async_copy(dst_hbm.at[pl.ds(base, CH)], didxs[b], sems[b])

        def Lfirst(i, b):
            base = w * EPW + i * CH
            pltpu.async_copy(e_hbm.at[pl.ds(base, CH)], ebufs[b], sems[b])
            pltpu.async_copy(src_hbm.at[pl.ds(base, CH)], sidxs[b], sems[b])
            pltpu.async_copy(dst_hbm.at[pl.ds(base, CH)], didxs[b], sems[b])

        def G(i, b):
            base = w * EPW + i * CH
            pltpu.make_async_copy(e_hbm.at[pl.ds(base, CH)],
                                  ebufs[b], sems[b]).wait()
            pltpu.make_async_copy(src_hbm.at[pl.ds(base, CH)],
                                  sidxs[b], sems[b]).wait()
            pltpu.make_async_copy(dst_hbm.at[pl.ds(base, CH)],
                                  didxs[b], sems[b]).wait()
            pltpu.async_copy(h_hbm.at[sidxs[b]], ebufs[b], sems[b], add=True)

        def C(b):
            pltpu.make_async_copy(h_hbm.at[sidxs[b]], ebufs[b],
                                  sems[b]).wait()

            @pl.loop(0, CH)
            def _relu(r):
                for j in range(8):
                    v = ebufs[b][r, pl.ds(j * 16, 16)]
                    ebufs[b][r, pl.ds(j * 16, 16)] = jnp.maximum(v, 0.0)

            pltpu.async_copy(ebufs[b], aggr_sh.at[didxs[b]], sems[b],
                             add=True)

        Lfirst(0, 0)
        Lfirst(1, 1)
        G(0, 0)

        @pl.loop(0, NSTEP - 1, step=NB)
        def _outer(it):
            for o in range(NB):
                i = it + o

                @pl.when(i + 2 < NSTEP)
                def _(i=i, o=o):
                    L(i + 2, (o + 2) % NB, guard_scatter=True)

                G(i + 1, (o + 1) % NB)
                C(o)

        C((NSTEP - 1) % NB)
        for b in range(NB):
            pltpu.make_async_copy(ebufs[b], aggr_sh.at[didxs[b]],
                                  sems[b]).wait()

        plsc.subcore_barrier()
        pltpu.sync_copy(aggr_sh.at[pl.ds(s * RSUB, RSUB)],
                        out_hbm.at[c].at[pl.ds(s * RSUB, RSUB)])

        @pl.when(s == 0)
        def _():
            pltpu.sync_copy(aggr_sh.at[pl.ds(NS * RSUB, RTAIL)],
                            out_hbm.at[c].at[pl.ds(NS * RSUB, RTAIL)])

    return k(h, e, edge_index.reshape(2 * E), zeros)


def _node_block(h_ref, a_ref, w1_ref, b1_ref, w2_ref, b2_ref,
                gnw_ref, gnb_ref, gnms_ref, o_ref):
    h = h_ref[...]
    z0 = h + a_ref[0] + a_ref[1]
    t = jnp.maximum(
        jnp.dot(z0, w1_ref[...], preferred_element_type=jnp.float32)
        + b1_ref[...], 0.0)
    t = jnp.dot(t, w2_ref[...], preferred_element_type=jnp.float32) + b2_ref[...]
    mean = jnp.mean(t, axis=0, keepdims=True)
    cen = t - gnms_ref[...] * mean
    var = jnp.mean(cen * cen, axis=0, keepdims=True)
    zn = gnw_ref[...] * cen * lax.rsqrt(var + 1e-5) + gnb_ref[...]
    o_ref[...] = jnp.maximum(zn, 0.0) + h


def _node_update(h, aggr2, lp):
    return pl.pallas_call(
        _node_block,
        out_shape=jax.ShapeDtypeStruct((N, H), jnp.float32),
    )(h, aggr2,
      lp["W1"], lp["b1"].reshape(1, H), lp["W2"], lp["b2"].reshape(1, H),
      lp["gn_w"].reshape(1, H), lp["gn_b"].reshape(1, H),
      lp["gn_ms"].reshape(1, H))


def _head_block(z1_ref, z2_ref, z3_ref, wh1_ref, bh1_ref, wh2_ref, bh2_ref,
                oa_ref, ob_ref):
    w = wh1_ref[...]
    t = (jnp.dot(z1_ref[...], w[0:H], preferred_element_type=jnp.float32)
         + jnp.dot(z2_ref[...], w[H:2 * H], preferred_element_type=jnp.float32)
         + jnp.dot(z3_ref[...], w[2 * H:3 * H],
                   preferred_element_type=jnp.float32))
    t = jnp.maximum(t + bh1_ref[...], 0.0)
    o = (jnp.dot(t, wh2_ref[...], preferred_element_type=jnp.float32)
         + bh2_ref[...])
    oa_ref[...] = o[:, :SPLIT]
    ob_ref[...] = o[:, SPLIT:OUT]


def _head(z1, z2, z3, Wh1, bh1, Wh2, bh2):
    wh2p = jnp.zeros((H, 8), jnp.float32).at[:, :OUT].set(Wh2)
    bh2p = jnp.zeros((1, 8), jnp.float32).at[0, :OUT].set(bh2)
    return pl.pallas_call(
        _head_block,
        out_shape=(jax.ShapeDtypeStruct((N, SPLIT), jnp.float32),
                   jax.ShapeDtypeStruct((N, OUT - SPLIT), jnp.float32)),
    )(z1, z2, z3, Wh1, bh1.reshape(1, H), wh2p, bh2p)


def kernel(x, edge_index, edge_attr, params):
    zeros = jnp.zeros((N, H), jnp.float32)

    es = [_edge_mlp(edge_attr, lp["We"], lp["be"]) for lp in params["layers"]]

    h = x
    outs = []
    for l, lp in enumerate(params["layers"]):
        aggr2 = _sc_aggregate(h, es[l], edge_index, zeros)
        h = _node_update(h, aggr2, lp)
        outs.append(h)

    return _head(outs[0], outs[1], outs[2],
                 params["Wh1"], params["bh1"], params["Wh2"], params["bh2"])
